# baseline (speedup 1.0000x reference)
import jax
import jax.numpy as jnp
from jax.experimental import pallas as pl
from jax.experimental.pallas import tpu as pltpu

NB = 96
H = 224
W = 224
C_BLK = 12
B_BLK = 1


def _onehot_body(x_ref, o_ref):
    c0 = pl.program_id(1) * C_BLK
    cls = c0 + jax.lax.broadcasted_iota(jnp.int32, (C_BLK, H, W), 0)
    for b in range(B_BLK):
        o_ref[b] = (x_ref[b, 0][None, :, :] == cls).astype(jnp.float32)


def kernel(x):
    grid = (x.shape[0] // B_BLK, NB // C_BLK)
    return pl.pallas_call(
        _onehot_body,
        grid=grid,
        in_specs=[pl.BlockSpec((B_BLK, 1, H, W), lambda b, c: (b, 0, 0, 0))],
        out_specs=pl.BlockSpec((B_BLK, C_BLK, H, W), lambda b, c: (b, c, 0, 0)),
        out_shape=jax.ShapeDtypeStruct((x.shape[0], NB, H, W), jnp.float32),
        compiler_params=pltpu.CompilerParams(
            dimension_semantics=("parallel", "parallel"),
        ),
    )(x)


# speedup vs baseline: 1.1557x; 1.1557x over previous
import jax
import jax.numpy as jnp
from jax.experimental import pallas as pl
from jax.experimental.pallas import tpu as pltpu

NB = 96
H = 224
W = 224
C_BLK = 32
B_BLK = 1


def _onehot_body(x_ref, o_ref):
    c0 = pl.program_id(1) * C_BLK
    cls = c0 + jax.lax.broadcasted_iota(jnp.int32, (C_BLK, H, W), 0)
    for b in range(B_BLK):
        o_ref[b] = (x_ref[b, 0][None, :, :] == cls).astype(jnp.float32)


def kernel(x):
    grid = (x.shape[0] // B_BLK, NB // C_BLK)
    return pl.pallas_call(
        _onehot_body,
        grid=grid,
        in_specs=[pl.BlockSpec((B_BLK, 1, H, W), lambda b, c: (b, 0, 0, 0))],
        out_specs=pl.BlockSpec((B_BLK, C_BLK, H, W), lambda b, c: (b, c, 0, 0)),
        out_shape=jax.ShapeDtypeStruct((x.shape[0], NB, H, W), jnp.float32),
        compiler_params=pltpu.CompilerParams(
            dimension_semantics=("parallel", "parallel"),
        ),
    )(x)
